# baseline (speedup 1.0000x reference)
import functools

import jax
import jax.numpy as jnp
import numpy as np
from jax import lax
from jax.experimental import pallas as pl
from jax.experimental.pallas import tpu as pltpu
from jax.experimental.pallas import tpu_sc as plsc

INPUT_DIM = 4096
K = 1024
TEMP = 5.0
ROWS = 4 * 2048
BLK = 512
NCHUNK = INPUT_DIM // 16
EXT = 512

def _soft_probs(logits):
    u = jnp.clip(
        jax.random.uniform(jax.random.key(1), logits.shape, dtype=jnp.float32),
        1e-06,
        None,
    )
    noise = -jnp.log(-jnp.log(u) + 1e-06)
    return jax.nn.sigmoid((logits + noise) / TEMP)


def _sc_kth_kernel(sp_hbm, out_hbm, sp_v, ext_v, out_v):
    cid = lax.axis_index("c")
    sid = lax.axis_index("s")

    @pl.when(jnp.logical_and(cid == 0, sid == 0))
    def _():
        pltpu.sync_copy(sp_hbm, sp_v)
        k_v = jnp.full((16,), K, jnp.int32)
        ones_i = jnp.ones((16,), jnp.int32)
        zeros_i = jnp.zeros((16,), jnp.int32)
        fifteen = jnp.full((16,), 15, jnp.int32)

        UNROLL = 8

        def splat_total(acc):
            return plsc.cumsum(acc).at[fifteen].get(mode="promise_in_bounds")

        def count_ge(t_v):
            def body(i, acc):
                for u in range(UNROLL):
                    acc = acc + jnp.where(
                        sp_v[pl.ds((i * UNROLL + u) * 16, 16)] >= t_v,
                        ones_i,
                        zeros_i,
                    )
                return acc

            acc = lax.fori_loop(
                0, NCHUNK // UNROLL, body, jnp.zeros((16,), jnp.int32)
            )
            return splat_total(acc)

        def nbody(_, carry):
            lo, hi, cnt_hi = carry
            mid = 0.5 * (lo + hi)
            cnt = count_ge(mid)
            big = cnt >= k_v
            return (
                jnp.where(big, mid, lo),
                jnp.where(big, hi, mid),
                jnp.where(big, cnt_hi, cnt),
            )

        lo, hi, cnt_hi = lax.fori_loop(
            0, 6, nbody,
            (
                jnp.zeros((16,), jnp.float32),
                jnp.ones((16,), jnp.float32),
                jnp.zeros((16,), jnp.int32),
            ),
        )
        r_v = k_v - cnt_hi

        for i in range(EXT // 16):
            ext_v[pl.ds(i * 16, 16)] = jnp.full((16,), -1.0, jnp.float32)

        ZONE = EXT // 4
        S = NCHUNK // 4

        def ebody(i, offs):
            new_offs = []
            for s in range(4):
                off = offs[s]
                v = sp_v[pl.ds((s * S + i) * 16, 16)]
                m = jnp.logical_and(v >= lo, v < hi)
                mi = jnp.where(m, ones_i, zeros_i)
                cs = plsc.cumsum(mi)
                idx = jnp.minimum(off, s * ZONE + ZONE - 16) + (cs - mi)
                plsc.store_scatter(ext_v, [idx], v, mask=m)
                new_offs.append(
                    off + cs.at[fifteen].get(mode="promise_in_bounds")
                )
            return tuple(new_offs)

        lax.fori_loop(
            0, S, ebody,
            tuple(jnp.full((16,), s * ZONE, jnp.int32) for s in range(4)),
        )

        def kcond(bounds):
            blo, bhi = bounds
            return jnp.any(blo < bhi)

        def kbody(bounds):
            blo, bhi = bounds
            mid = blo + (bhi - blo + 1) // 2
            t_v = plsc.bitcast(mid, jnp.float32)

            def cbody(i, acc):
                for u in range(UNROLL):
                    acc = acc + jnp.where(
                        ext_v[pl.ds((i * UNROLL + u) * 16, 16)] >= t_v,
                        ones_i,
                        zeros_i,
                    )
                return acc

            acc = lax.fori_loop(
                0, EXT // 16 // UNROLL, cbody, jnp.zeros((16,), jnp.int32)
            )
            big = splat_total(acc) >= r_v
            return (jnp.where(big, mid, blo), jnp.where(big, bhi, mid - 1))

        blo = plsc.bitcast(lo, jnp.int32)
        bhi = plsc.bitcast(hi, jnp.int32)
        blo, bhi = lax.while_loop(kcond, kbody, (blo, bhi))

        out_v[...] = plsc.bitcast(blo, jnp.float32)
        pltpu.sync_copy(out_v, out_hbm)


_sc_kth = pl.kernel(
    _sc_kth_kernel,
    out_type=jax.ShapeDtypeStruct((16,), jnp.float32),
    compiler_params=pltpu.CompilerParams(needs_layout_passes=False),
    mesh=plsc.VectorSubcoreMesh(core_axis_name="c", subcore_axis_name="s"),
    scratch_types=[
        pltpu.VMEM((INPUT_DIM,), jnp.float32),
        pltpu.VMEM((EXT,), jnp.float32),
        pltpu.VMEM((16,), jnp.float32),
    ],
)


def _mask_mul_kernel(x_ref, sp_ref, kth_ref, o_ref):
    mask = (sp_ref[...] >= kth_ref[0, 0]).astype(jnp.float32)
    o_ref[...] = x_ref[...] * mask


@jax.jit
def kernel(x, logits):
    sp = _soft_probs(logits)

    kth16 = _sc_kth(sp)

    x2d = x.reshape(ROWS, INPUT_DIM)
    out = pl.pallas_call(
        _mask_mul_kernel,
        grid=(ROWS // BLK,),
        in_specs=[
            pl.BlockSpec((BLK, INPUT_DIM), lambda i: (i, 0)),
            pl.BlockSpec((1, INPUT_DIM), lambda i: (0, 0)),
            pl.BlockSpec((1, 16), lambda i: (0, 0)),
        ],
        out_specs=pl.BlockSpec((BLK, INPUT_DIM), lambda i: (i, 0)),
        out_shape=jax.ShapeDtypeStruct((ROWS, INPUT_DIM), jnp.float32),
        compiler_params=pltpu.CompilerParams(
            dimension_semantics=("arbitrary",),
        ),
    )(x2d, sp.reshape(1, INPUT_DIM), kth16.reshape(1, 16))
    return out.reshape(x.shape)


# speedup vs baseline: 1.2205x; 1.2205x over previous
import jax
import jax.numpy as jnp
from jax.experimental import pallas as pl
from jax.experimental.pallas import tpu as pltpu

INPUT_DIM = 4096
K = 1024
TEMP = 5.0
ROWS = 4 * 2048
BLK = 512


def _soft_probs(logits):
    u = jnp.clip(
        jax.random.uniform(jax.random.key(1), logits.shape, dtype=jnp.float32),
        1e-06,
        None,
    )
    noise = -jnp.log(-jnp.log(u) + 1e-06)
    return jax.nn.sigmoid((logits + noise) / TEMP)


def _mask_mul_kernel(x_ref, sp_ref, o_ref, kth_smem):
    sp = sp_ref[...]

    @pl.when(pl.program_id(0) == 0)
    def _find_kth():
        def body(_, carry):
            lo, hi = carry
            mid = lo + (hi - lo + 1) // 2
            cnt = jnp.sum(
                (sp >= jax.lax.bitcast_convert_type(mid, jnp.float32)).astype(
                    jnp.int32
                )
            )
            big = cnt >= K
            return (jnp.where(big, mid, lo), jnp.where(big, hi, mid - 1))

        lo = jnp.int32(0)
        hi = jnp.int32(0x3F800000)
        lo, hi = jax.lax.fori_loop(0, 31, body, (lo, hi))
        kth_smem[0] = lo

    kth = jax.lax.bitcast_convert_type(kth_smem[0], jnp.float32)
    mask = (sp >= kth).astype(jnp.float32)
    o_ref[...] = x_ref[...] * mask


@jax.jit
def kernel(x, logits):
    sp = _soft_probs(logits).reshape(1, INPUT_DIM)

    x2d = x.reshape(ROWS, INPUT_DIM)
    out = pl.pallas_call(
        _mask_mul_kernel,
        grid=(ROWS // BLK,),
        in_specs=[
            pl.BlockSpec((BLK, INPUT_DIM), lambda i: (i, 0)),
            pl.BlockSpec((1, INPUT_DIM), lambda i: (0, 0)),
        ],
        out_specs=pl.BlockSpec((BLK, INPUT_DIM), lambda i: (i, 0)),
        out_shape=jax.ShapeDtypeStruct((ROWS, INPUT_DIM), jnp.float32),
        scratch_shapes=[pltpu.SMEM((1,), jnp.int32)],
        compiler_params=pltpu.CompilerParams(
            dimension_semantics=("arbitrary",),
        ),
    )(x2d, sp)
    return out.reshape(x.shape)


# speedup vs baseline: 1.2389x; 1.0151x over previous
import jax
import jax.numpy as jnp
import numpy as np
from jax.experimental import pallas as pl
from jax.experimental.pallas import tpu as pltpu

INPUT_DIM = 4096
K = 1024
TEMP = 5.0
ROWS = 4 * 2048
BLK = 512


def _np_threefry_uniform(n):
    k0, k1 = np.uint32(0), np.uint32(1)
    ks = [k0, k1, np.uint32(k0 ^ k1 ^ np.uint32(0x1BD11BDA))]
    with np.errstate(over="ignore"):
        x0 = np.zeros(n, dtype=np.uint32) + ks[0]
        x1 = (np.arange(n, dtype=np.uint32) + ks[1]).astype(np.uint32)

        def rotl(x, d):
            return ((x << np.uint32(d)) | (x >> np.uint32(32 - d))).astype(
                np.uint32
            )

        rotations = [(13, 15, 26, 6), (17, 29, 16, 24)]
        for i in range(5):
            for r in rotations[i % 2]:
                x0 = (x0 + x1).astype(np.uint32)
                x1 = rotl(x1, r)
                x1 = (x1 ^ x0).astype(np.uint32)
            x0 = (x0 + ks[(i + 1) % 3]).astype(np.uint32)
            x1 = (x1 + ks[(i + 2) % 3] + np.uint32(i + 1)).astype(np.uint32)
        bits = (x0 ^ x1).astype(np.uint32)
    u = (((bits >> np.uint32(9)) | np.uint32(0x3F800000)).view(np.float32)
         - np.float32(1.0))
    return u


_U = np.maximum(_np_threefry_uniform(INPUT_DIM), np.float32(1e-06)).reshape(
    1, INPUT_DIM
)


def _mask_mul_kernel(x_ref, lg_ref, u_ref, o_ref, kth_smem):
    noise = -jnp.log(-jnp.log(u_ref[...]) + 1e-06)
    sp = jax.nn.sigmoid((lg_ref[...] + noise) / TEMP)

    @pl.when(pl.program_id(0) == 0)
    def _find_kth():
        def body(_, carry):
            lo, hi = carry
            mid = lo + (hi - lo + 1) // 2
            cnt = jnp.sum(
                (sp >= jax.lax.bitcast_convert_type(mid, jnp.float32)).astype(
                    jnp.int32
                )
            )
            big = cnt >= K
            return (jnp.where(big, mid, lo), jnp.where(big, hi, mid - 1))

        lo = jnp.int32(0)
        hi = jnp.int32(0x3F800000)
        lo, hi = jax.lax.fori_loop(0, 31, body, (lo, hi))
        kth_smem[0] = lo

    kth = jax.lax.bitcast_convert_type(kth_smem[0], jnp.float32)
    mask = (sp >= kth).astype(jnp.float32)
    o_ref[...] = x_ref[...] * mask


@jax.jit
def kernel(x, logits):
    lg = logits.reshape(1, INPUT_DIM)
    u = jnp.asarray(_U)

    x2d = x.reshape(ROWS, INPUT_DIM)
    out = pl.pallas_call(
        _mask_mul_kernel,
        grid=(ROWS // BLK,),
        in_specs=[
            pl.BlockSpec((BLK, INPUT_DIM), lambda i: (i, 0)),
            pl.BlockSpec((1, INPUT_DIM), lambda i: (0, 0)),
            pl.BlockSpec((1, INPUT_DIM), lambda i: (0, 0)),
        ],
        out_specs=pl.BlockSpec((BLK, INPUT_DIM), lambda i: (i, 0)),
        out_shape=jax.ShapeDtypeStruct((ROWS, INPUT_DIM), jnp.float32),
        scratch_shapes=[pltpu.SMEM((1,), jnp.int32)],
        compiler_params=pltpu.CompilerParams(
            dimension_semantics=("arbitrary",),
        ),
    )(x2d, lg, u)
    return out.reshape(x.shape)
